# baseline (speedup 1.0000x reference)
import functools

import jax
import jax.numpy as jnp
from jax.experimental import pallas as pl
from jax.experimental.pallas import tpu as pltpu


def _cdiv(a, b):
    return -(-a // b)


def _ffn_kernel(x_ref, w1_ref, b1_ref, w2_ref, b2_ref, o_ref):
    h = jnp.dot(x_ref[...], w1_ref[...], preferred_element_type=jnp.float32)
    h = jnp.maximum(h + b1_ref[...], 0.0)
    y = jnp.dot(h.astype(jnp.bfloat16), w2_ref[...],
                preferred_element_type=jnp.float32)
    o_ref[...] = y + b2_ref[...]


@functools.partial(jax.jit, static_argnames=("block_m",))
def _ffn(x, w1, b1, w2, b2, *, block_m=512):
    batch, seq, d_model = x.shape
    hidden = w1.shape[1]
    M = batch * seq

    x2d = x.reshape(M, d_model).astype(jnp.bfloat16)
    w1b = w1.astype(jnp.bfloat16)
    w2b = w2.astype(jnp.bfloat16)

    bm = min(block_m, M)
    n_m = _cdiv(M, bm)

    out2d = pl.pallas_call(
        _ffn_kernel,
        out_shape=jax.ShapeDtypeStruct((M, d_model), jnp.float32),
        grid=(n_m,),
        in_specs=[
            pl.BlockSpec((bm, d_model), lambda i: (i, 0)),
            pl.BlockSpec((d_model, hidden), lambda i: (0, 0)),
            pl.BlockSpec((1, hidden), lambda i: (0, 0)),
            pl.BlockSpec((hidden, d_model), lambda i: (0, 0)),
            pl.BlockSpec((1, d_model), lambda i: (0, 0)),
        ],
        out_specs=pl.BlockSpec((bm, d_model), lambda i: (i, 0)),
        compiler_params=pltpu.CompilerParams(
            dimension_semantics=("parallel",),
            vmem_limit_bytes=int(0.9 * 64 * 1024 * 1024),
        ),
    )(x2d, w1b, b1, w2b, b2)

    return out2d.reshape(batch, seq, d_model)


def kernel(x, w1, b1, w2, b2):
    return _ffn(x, w1, b1, w2, b2)


# speedup vs baseline: 1.2438x; 1.2438x over previous
import functools

import jax
import jax.numpy as jnp
from jax.experimental import pallas as pl
from jax.experimental.pallas import tpu as pltpu


def _cdiv(a, b):
    return -(-a // b)


def _ffn_kernel(x_ref, w1_ref, b1_ref, w2_ref, b2_ref, o_ref):
    h = jnp.dot(x_ref[...], w1_ref[...], preferred_element_type=jnp.float32)
    h = jnp.maximum(h + b1_ref[...], 0.0)
    y = jnp.dot(h, w2_ref[...], preferred_element_type=jnp.float32)
    o_ref[...] = y + b2_ref[...]


@functools.partial(jax.jit, static_argnames=("block_m",))
def _ffn(x, w1, b1, w2, b2, *, block_m=512):
    batch, seq, d_model = x.shape
    hidden = w1.shape[1]
    M = batch * seq

    x2d = x.reshape(M, d_model)
    bm = min(block_m, M)
    n_m = _cdiv(M, bm)

    out2d = pl.pallas_call(
        _ffn_kernel,
        out_shape=jax.ShapeDtypeStruct((M, d_model), jnp.float32),
        grid=(n_m,),
        in_specs=[
            pl.BlockSpec((bm, d_model), lambda i: (i, 0)),
            pl.BlockSpec((d_model, hidden), lambda i: (0, 0)),
            pl.BlockSpec((1, hidden), lambda i: (0, 0)),
            pl.BlockSpec((hidden, d_model), lambda i: (0, 0)),
            pl.BlockSpec((1, d_model), lambda i: (0, 0)),
        ],
        out_specs=pl.BlockSpec((bm, d_model), lambda i: (i, 0)),
        compiler_params=pltpu.CompilerParams(
            dimension_semantics=("parallel",),
            vmem_limit_bytes=int(0.95 * 64 * 1024 * 1024),
        ),
    )(x2d, w1, b1, w2, b2)

    return out2d.reshape(batch, seq, d_model)


def kernel(x, w1, b1, w2, b2):
    return _ffn(x, w1, b1, w2, b2)


# speedup vs baseline: 1.2444x; 1.0004x over previous
import functools

import jax
import jax.numpy as jnp
from jax.experimental import pallas as pl
from jax.experimental.pallas import tpu as pltpu


def _cdiv(a, b):
    return -(-a // b)


def _ffn_kernel(x_ref, w1_ref, b1_ref, w2_ref, b2_ref, o_ref):
    h = jnp.dot(x_ref[...], w1_ref[...], preferred_element_type=jnp.float32)
    h = jnp.maximum(h + b1_ref[...], 0.0)
    y = jnp.dot(h, w2_ref[...], preferred_element_type=jnp.float32)
    o_ref[...] = y + b2_ref[...]


@functools.partial(jax.jit, static_argnames=("block_m",))
def _ffn(x, w1, b1, w2, b2, *, block_m=512):
    batch, seq, d_model = x.shape
    hidden = w1.shape[1]
    M = batch * seq

    x2d = x.reshape(M, d_model)
    bm = min(block_m, M)
    n_m = _cdiv(M, bm)

    out2d = pl.pallas_call(
        _ffn_kernel,
        out_shape=jax.ShapeDtypeStruct((M, d_model), jnp.float32),
        grid=(n_m,),
        in_specs=[
            pl.BlockSpec((bm, d_model), lambda i: (i, 0)),
            pl.BlockSpec((d_model, hidden), lambda i: (0, 0)),
            pl.BlockSpec((1, hidden), lambda i: (0, 0)),
            pl.BlockSpec((hidden, d_model), lambda i: (0, 0)),
            pl.BlockSpec((1, d_model), lambda i: (0, 0)),
        ],
        out_specs=pl.BlockSpec((bm, d_model), lambda i: (i, 0)),
        compiler_params=pltpu.CompilerParams(
            dimension_semantics=("arbitrary",),
            vmem_limit_bytes=int(0.95 * 64 * 1024 * 1024),
        ),
    )(x2d, w1, b1, w2, b2)

    return out2d.reshape(batch, seq, d_model)


def kernel(x, w1, b1, w2, b2):
    return _ffn(x, w1, b1, w2, b2)
